# baseline (speedup 1.0000x reference)
import functools

import jax
import jax.numpy as jnp
from jax import lax
from jax.experimental import pallas as pl
from jax.experimental.pallas import tpu as pltpu
from jax.experimental.pallas import tpu_sc as plsc

N = 10000
E = 160000
H = 256
BLK = 256
N_PAD = 10240
HH = 128
NT = 16
ROWS_PER_TILE = N_PAD // NT
E_PER_TILE = E // NT
CH = 80
NCH = E_PER_TILE // CH


def _sc_sum_body(h_lo, h_hi, src0, dst0, src1, dst1, zrows, zcnt,
                 s0l, s0h, s1l, s1h, c0, c1,
                 acc, cnt_acc, src_v, dst_v, rows_a, rows_b, ones_v,
                 sga, sgb):
    c = lax.axis_index("c")
    s = lax.axis_index("s")
    for i in range(CH // 16):
        ones_v[pl.ds(i * 16, 16)] = jnp.full((16,), 1.0, jnp.float32)

    def edge_loop(src_r, dst_r, h_tab, count):
        pltpu.sync_copy(src_r.at[pl.ds(s * E_PER_TILE, E_PER_TILE)], src_v)
        pltpu.sync_copy(dst_r.at[pl.ds(s * E_PER_TILE, E_PER_TILE)], dst_v)
        rows = (rows_a, rows_b)
        sems = (sga, sgb)
        pltpu.async_copy(h_tab.at[src_v.at[pl.ds(0, CH)]], rows_a, sga)

        def do(j, b):
            @pl.when(j + 1 < NCH)
            def _():
                pltpu.async_copy(
                    h_tab.at[src_v.at[pl.ds((j + 1) * CH, CH)]],
                    rows[1 - b], sems[1 - b])
            pltpu.make_async_copy(
                h_tab.at[src_v.at[pl.ds(j * CH, CH)]], rows[b],
                sems[b]).wait()
            pltpu.sync_copy(rows[b], acc.at[dst_v.at[pl.ds(j * CH, CH)]],
                            add=True)
            if count:
                pltpu.sync_copy(ones_v,
                                cnt_acc.at[dst_v.at[pl.ds(j * CH, CH)]],
                                add=True)

        def pair(jj, carry):
            do(jj * 2, 0)
            do(jj * 2 + 1, 1)
            return carry
        lax.fori_loop(0, NCH // 2, pair, 0)
        do(NCH - 1, 0)

    row0 = s * ROWS_PER_TILE
    for (src_r, dst_r, out_l, out_h, cnt_out) in (
            (src0, dst0, s0l, s0h, c0), (src1, dst1, s1l, s1h, c1)):
        pltpu.sync_copy(zrows, acc.at[pl.ds(row0, ROWS_PER_TILE)])

        @pl.when(c == 0)
        def _():
            pltpu.sync_copy(zcnt, cnt_acc.at[pl.ds(row0, ROWS_PER_TILE)])
        plsc.subcore_barrier()

        @pl.when(c == 0)
        def _():
            edge_loop(src_r, dst_r, h_lo, True)

        @pl.when(c == 1)
        def _():
            edge_loop(src_r, dst_r, h_hi, False)
        plsc.subcore_barrier()

        @pl.when(c == 0)
        def _():
            pltpu.sync_copy(acc.at[pl.ds(row0, ROWS_PER_TILE)],
                            out_l.at[pl.ds(row0, ROWS_PER_TILE)])
            pltpu.sync_copy(cnt_acc.at[pl.ds(row0, ROWS_PER_TILE)],
                            cnt_out.at[pl.ds(row0, ROWS_PER_TILE)])

        @pl.when(c == 1)
        def _():
            pltpu.sync_copy(acc.at[pl.ds(row0, ROWS_PER_TILE)],
                            out_h.at[pl.ds(row0, ROWS_PER_TILE)])
        plsc.subcore_barrier()


@functools.partial(jax.jit, donate_argnums=())
def _sc_sum(h_lo, h_hi, src0, dst0, src1, dst1):
    f32 = jnp.float32
    zrows = jnp.zeros((ROWS_PER_TILE, HH), f32)
    zcnt = jnp.zeros((ROWS_PER_TILE,), f32)
    out_type = [jax.ShapeDtypeStruct((N_PAD, HH), f32) for _ in range(4)] + \
               [jax.ShapeDtypeStruct((N_PAD,), f32) for _ in range(2)]
    scratch = [
        pltpu.VMEM_SHARED((N_PAD, HH), f32),
        pltpu.VMEM_SHARED((N_PAD,), f32),
        pltpu.VMEM((E_PER_TILE,), jnp.int32),
        pltpu.VMEM((E_PER_TILE,), jnp.int32),
        pltpu.VMEM((CH, HH), f32),
        pltpu.VMEM((CH, HH), f32),
        pltpu.VMEM((CH,), f32),
        pltpu.SemaphoreType.DMA,
        pltpu.SemaphoreType.DMA,
    ]
    fn = pl.kernel(
        _sc_sum_body,
        out_type=out_type,
        compiler_params=pltpu.CompilerParams(needs_layout_passes=False),
        mesh=plsc.VectorSubcoreMesh(core_axis_name="c", subcore_axis_name="s"),
        scratch_types=scratch,
    )
    return fn(h_lo, h_hi, src0, dst0, src1, dst1, zrows, zcnt)


def _h_kernel(x_ref, wt_ref, b_ref, h_ref):
    h_ref[...] = jax.nn.relu(
        jnp.dot(x_ref[...], wt_ref[...], preferred_element_type=jnp.float32)
        + b_ref[...]
    )


def _compute_h(x_pad, W_init, b_init):
    wt = W_init.T
    b2 = b_init.reshape(1, H)
    grid = (N_PAD // BLK,)
    return pl.pallas_call(
        _h_kernel,
        grid=grid,
        in_specs=[
            pl.BlockSpec((BLK, H), lambda i: (i, 0)),
            pl.BlockSpec((H, H), lambda i: (0, 0)),
            pl.BlockSpec((1, H), lambda i: (0, 0)),
        ],
        out_specs=pl.BlockSpec((BLK, H), lambda i: (i, 0)),
        out_shape=jax.ShapeDtypeStruct((N_PAD, H), jnp.float32),
    )(x_pad, wt, b2)


def _final_kernel(h_ref, s0l_ref, s0h_ref, m0_ref, s1l_ref, s1h_ref, m1_ref,
                  c0_ref, c1_ref,
                  wlm0_ref, wrm0_ref, wlx0_ref, wrx0_ref,
                  wlm1_ref, wrm1_ref, wlx1_ref, wrx1_ref,
                  wadj_ref, wp0_ref, wp1_ref, wp2_ref, wp3_ref,
                  blm0_ref, blx0_ref, blm1_ref, blx1_ref,
                  badj_ref, bpost_ref, out_ref):
    h = h_ref[...]
    f32 = jnp.float32

    def dot(a, b):
        return jnp.dot(a, b, preferred_element_type=f32)

    inv0 = 1.0 / jnp.maximum(c0_ref[...], 1.0)
    inv1 = 1.0 / jnp.maximum(c1_ref[...], 1.0)

    def mean_dot(sl_ref, sh_ref, inv, w_ref):
        return (dot(sl_ref[...] * inv, w_ref[:HH, :])
                + dot(sh_ref[...] * inv, w_ref[HH:, :]))

    a0 = jax.nn.relu(mean_dot(s0l_ref, s0h_ref, inv0, wlm0_ref)
                     + blm0_ref[...] + dot(h, wrm0_ref[...]))
    b0 = jax.nn.relu(dot(m0_ref[...].astype(f32), wlx0_ref[...])
                     + blx0_ref[...] + dot(h, wrx0_ref[...]))
    a1 = jax.nn.relu(mean_dot(s1l_ref, s1h_ref, inv1, wlm1_ref)
                     + blm1_ref[...] + dot(h, wrm1_ref[...]))
    b1 = jax.nn.relu(dot(m1_ref[...].astype(f32), wlx1_ref[...])
                     + blx1_ref[...] + dot(h, wrx1_ref[...]))

    wadj = wadj_ref[...]
    badj = badj_ref[...]
    ym0 = dot(a0, wadj) + badj
    yx0 = dot(b0, wadj) + badj
    ym1 = dot(a1, wadj) + badj
    yx1 = dot(b1, wadj) + badj

    logits = (dot(ym0, wp0_ref[...]) + dot(yx0, wp1_ref[...])
              + dot(ym1, wp2_ref[...]) + dot(yx1, wp3_ref[...])
              + bpost_ref[...])

    mx = jnp.max(logits, axis=1, keepdims=True)
    z = logits - mx
    lse = jnp.log(jnp.sum(jnp.exp(z), axis=1, keepdims=True))
    out_ref[...] = z - lse


def _final_stage(h_pad, s0l, s0h, m0, s1l, s1h, m1, c0, c1,
                 Wl_mean0, bl_mean0, Wr_mean0, Wl_max0, bl_max0, Wr_max0,
                 Wl_mean1, bl_mean1, Wr_mean1, Wl_max1, bl_max1, Wr_max1,
                 W_adj, b_adj, W_post, b_post):
    grid = (N_PAD // BLK,)
    row_spec = pl.BlockSpec((BLK, H), lambda i: (i, 0))
    half_spec = pl.BlockSpec((BLK, HH), lambda i: (i, 0))
    col_spec = pl.BlockSpec((BLK, 1), lambda i: (i, 0))
    w_spec = pl.BlockSpec((H, H), lambda i: (0, 0))
    b_spec = pl.BlockSpec((1, H), lambda i: (0, 0))
    wp = [W_post[:, k * H:(k + 1) * H].T for k in range(4)]
    return pl.pallas_call(
        _final_kernel,
        grid=grid,
        in_specs=([row_spec, half_spec, half_spec, row_spec,
                   half_spec, half_spec, row_spec]
                  + [col_spec] * 2 + [w_spec] * 13 + [b_spec] * 6),
        out_specs=row_spec,
        out_shape=jax.ShapeDtypeStruct((N_PAD, H), jnp.float32),
    )(h_pad, s0l, s0h, m0, s1l, s1h, m1,
      c0.reshape(N_PAD, 1), c1.reshape(N_PAD, 1),
      Wl_mean0.T, Wr_mean0.T, Wl_max0.T, Wr_max0.T,
      Wl_mean1.T, Wr_mean1.T, Wl_max1.T, Wr_max1.T,
      W_adj.T, wp[0], wp[1], wp[2], wp[3],
      bl_mean0.reshape(1, H), bl_max0.reshape(1, H),
      bl_mean1.reshape(1, H), bl_max1.reshape(1, H),
      b_adj.reshape(1, H), b_post.reshape(1, H))


NW = 32
RNG = N_PAD // NW
TRASH = RNG
HW = 128
SCAN_C = 8000
N_CHUNK = E // SCAN_C
GC = 128
CM_CAP = SCAN_C + GC


def _sc_max_body(hw_hbm, src0, dst0, src1, dst1, zmax,
                 m0, m1,
                 acc, dbuf0, sbuf0, dbuf1, sbuf1, cm_src, cm_dst,
                 rows0, rows1, sc0, sc1, sg0, sg1):
    i32 = jnp.int32
    c = lax.axis_index("c")
    s = lax.axis_index("s")
    w = s * 2 + c
    lo = w * RNG
    iota = lax.iota(i32, 16)
    zeros16 = jnp.zeros((16,), i32)
    dbufs, sbufs, scs = (dbuf0, dbuf1), (sbuf0, sbuf1), (sc0, sc1)
    rows, sgs = (rows0, rows1), (sg0, sg1)

    def init_cm(i, _):
        cm_src[pl.ds(i * 16, 16)] = zeros16
        return 0
    lax.fori_loop(0, CM_CAP // 16, init_cm, 0)

    def scan_group(g, nm_vec, db, sb):
        dv = db[pl.ds(g * 16, 16)]
        sv = sb[pl.ds(g * 16, 16)]
        dl = dv - lo
        m = (dl >= 0) & (dl < RNG)
        incl = plsc.cumsum(m.astype(i32))
        pos = nm_vec + incl - 1
        plsc.store_scatter(cm_src, [pos], sv, mask=m)
        plsc.store_scatter(cm_dst, [pos], dl, mask=m)
        return nm_vec + plsc.all_reduce_population_count(m)

    def rmw_group(rows_v, off2):
        dlv = cm_dst[pl.ds(off2, 16)]
        for k in range(16):
            dl = dlv[k]
            e = off2 % GC + k
            for f in range(HW // 16):
                a = acc[dl, pl.ds(f * 16, 16)]
                mg = rows_v[e, pl.ds(f * 16, 16)]
                r = jnp.maximum(plsc.bitcast(a, jnp.bfloat16),
                                plsc.bitcast(mg, jnp.bfloat16))
                acc[dl, pl.ds(f * 16, 16)] = plsc.bitcast(r, i32)

    def issue_scan(j, cb, dst_r, src_r):
        off = j * SCAN_C
        pltpu.async_copy(dst_r.at[pl.ds(off, SCAN_C)], dbufs[cb], scs[cb])
        pltpu.async_copy(src_r.at[pl.ds(off, SCAN_C)], sbufs[cb], scs[cb])

    def issue_gather(s2, rb):
        pltpu.async_copy(hw_hbm.at[cm_src.at[pl.ds(s2 * GC, GC)]],
                         rows[rb], sgs[rb])

    def layer(src_r, dst_r, out_r):
        pltpu.sync_copy(zmax, acc)
        issue_scan(0, 0, dst_r, src_r)

        def chunk_do(j, cb):
            off = j * SCAN_C
            pltpu.make_async_copy(dst_r.at[pl.ds(off, SCAN_C)], dbufs[cb],
                                  scs[cb]).wait()
            pltpu.make_async_copy(src_r.at[pl.ds(off, SCAN_C)], sbufs[cb],
                                  scs[cb]).wait()

            @pl.when(j + 1 < N_CHUNK)
            def _():
                issue_scan(j + 1, 1 - cb, dst_r, src_r)

            def grp4(jj, nm_vec):
                for k in range(4):
                    nm_vec = scan_group(jj * 4 + k, nm_vec, dbufs[cb],
                                        sbufs[cb])
                return nm_vec
            nm_vec = lax.fori_loop(0, SCAN_C // 64, grp4,
                                   jnp.zeros((16,), i32))
            nm = jnp.max(nm_vec)
            trash_splat = jnp.full((16,), TRASH, i32)
            plsc.store_scatter(cm_dst, [nm + iota], trash_splat)
            nm16 = (nm + 15) // 16 * 16
            n_sub = (nm16 + GC - 1) // GC

            def sub(s2, _):
                pltpu.async_copy(
                    hw_hbm.at[cm_src.at[pl.ds(s2 * GC, GC)]], rows0,
                    sg0).wait()
                ng = jnp.minimum(GC // 16, (nm16 - s2 * GC) // 16)

                def grp(gg, _):
                    rmw_group(rows0, s2 * GC + gg * 16)
                    return 0
                lax.fori_loop(0, ng, grp, 0)
                return 0
            lax.fori_loop(0, n_sub, sub, 0)
            return 0

        def cpair(jj, _):
            chunk_do(jj * 2, 0)
            chunk_do(jj * 2 + 1, 1)
            return 0
        lax.fori_loop(0, N_CHUNK // 2, cpair, 0)
        pltpu.sync_copy(acc.at[pl.ds(0, RNG)], out_r.at[pl.ds(lo, RNG)])

    layer(src0, dst0, m0)
    layer(src1, dst1, m1)


def _sc_max(hw, src0, dst0, src1, dst1):
    i32 = jnp.int32
    zmax = jnp.zeros((RNG + 1, HW), i32)
    out_type = [jax.ShapeDtypeStruct((N_PAD, HW), i32) for _ in range(2)]
    scratch = [
        pltpu.VMEM((RNG + 1, HW), i32),
        pltpu.VMEM((SCAN_C,), i32),
        pltpu.VMEM((SCAN_C,), i32),
        pltpu.VMEM((SCAN_C,), i32),
        pltpu.VMEM((SCAN_C,), i32),
        pltpu.VMEM((CM_CAP,), i32),
        pltpu.VMEM((CM_CAP,), i32),
        pltpu.VMEM((GC, HW), i32),
        pltpu.VMEM((GC, HW), i32),
        pltpu.SemaphoreType.DMA,
        pltpu.SemaphoreType.DMA,
        pltpu.SemaphoreType.DMA,
        pltpu.SemaphoreType.DMA,
    ]
    fn = pl.kernel(
        _sc_max_body,
        out_type=out_type,
        compiler_params=pltpu.CompilerParams(needs_layout_passes=False),
        mesh=plsc.VectorSubcoreMesh(core_axis_name="c", subcore_axis_name="s"),
        scratch_types=scratch,
    )
    return fn(hw, src0, dst0, src1, dst1, zmax)


def kernel(x, edge_index0, edge_index1, W_init, b_init,
           Wl_mean0, bl_mean0, Wr_mean0, Wl_max0, bl_max0, Wr_max0,
           Wl_mean1, bl_mean1, Wr_mean1, Wl_max1, bl_max1, Wr_max1,
           W_adj, b_adj, W_post, b_post):
    x_pad = jnp.pad(x, ((0, N_PAD - N), (0, 0)))
    h_pad = _compute_h(x_pad, W_init, b_init)
    h_lo = h_pad[:, :HH] + 0.0
    h_hi = h_pad[:, HH:] + 0.0
    hw = jax.lax.bitcast_convert_type(
        h_pad.astype(jnp.bfloat16).reshape(N_PAD, HW, 2), jnp.int32)
    s0l, s0h, s1l, s1h, c0, c1 = _sc_sum(
        h_lo, h_hi, edge_index0[0], edge_index0[1],
        edge_index1[0], edge_index1[1])
    m0w, m1w = _sc_max(hw, edge_index0[0], edge_index0[1],
                       edge_index1[0], edge_index1[1])
    m0 = jax.lax.bitcast_convert_type(m0w, jnp.bfloat16).reshape(N_PAD, H)
    m1 = jax.lax.bitcast_convert_type(m1w, jnp.bfloat16).reshape(N_PAD, H)
    out = _final_stage(h_pad, s0l, s0h, m0, s1l, s1h, m1, c0, c1,
                       Wl_mean0, bl_mean0, Wr_mean0, Wl_max0, bl_max0, Wr_max0,
                       Wl_mean1, bl_mean1, Wr_mean1, Wl_max1, bl_max1, Wr_max1,
                       W_adj, b_adj, W_post, b_post)
    return out[:N]


# speedup vs baseline: 1.6615x; 1.6615x over previous
import functools

import jax
import jax.numpy as jnp
from jax import lax
from jax.experimental import pallas as pl
from jax.experimental.pallas import tpu as pltpu
from jax.experimental.pallas import tpu_sc as plsc

N = 10000
E = 160000
H = 256
BLK = 256
N_PAD = 10240
HH = 128
NT = 16
ROWS_PER_TILE = N_PAD // NT
E_PER_TILE = E // NT
CH = 80
NCH = E_PER_TILE // CH


def _sc_sum_body(h_lo, h_hi, src0, dst0, src1, dst1, zrows, zcnt,
                 s0l, s0h, s1l, s1h, c0, c1,
                 acc, cnt_acc, src_v, dst_v, rows_a, rows_b, ones_v,
                 sga, sgb):
    c = lax.axis_index("c")
    s = lax.axis_index("s")
    for i in range(CH // 16):
        ones_v[pl.ds(i * 16, 16)] = jnp.full((16,), 1.0, jnp.float32)

    def edge_loop(src_r, dst_r, h_tab, count):
        pltpu.sync_copy(src_r.at[pl.ds(s * E_PER_TILE, E_PER_TILE)], src_v)
        pltpu.sync_copy(dst_r.at[pl.ds(s * E_PER_TILE, E_PER_TILE)], dst_v)
        rows = (rows_a, rows_b)
        sems = (sga, sgb)
        pltpu.async_copy(h_tab.at[src_v.at[pl.ds(0, CH)]], rows_a, sga)

        def do(j, b):
            @pl.when(j + 1 < NCH)
            def _():
                pltpu.async_copy(
                    h_tab.at[src_v.at[pl.ds((j + 1) * CH, CH)]],
                    rows[1 - b], sems[1 - b])
            pltpu.make_async_copy(
                h_tab.at[src_v.at[pl.ds(j * CH, CH)]], rows[b],
                sems[b]).wait()
            pltpu.sync_copy(rows[b], acc.at[dst_v.at[pl.ds(j * CH, CH)]],
                            add=True)
            if count:
                pltpu.sync_copy(ones_v,
                                cnt_acc.at[dst_v.at[pl.ds(j * CH, CH)]],
                                add=True)

        def pair(jj, carry):
            do(jj * 2, 0)
            do(jj * 2 + 1, 1)
            return carry
        lax.fori_loop(0, NCH // 2, pair, 0)
        do(NCH - 1, 0)

    row0 = s * ROWS_PER_TILE
    for (src_r, dst_r, out_l, out_h, cnt_out) in (
            (src0, dst0, s0l, s0h, c0), (src1, dst1, s1l, s1h, c1)):
        pltpu.sync_copy(zrows, acc.at[pl.ds(row0, ROWS_PER_TILE)])

        @pl.when(c == 0)
        def _():
            pltpu.sync_copy(zcnt, cnt_acc.at[pl.ds(row0, ROWS_PER_TILE)])
        plsc.subcore_barrier()

        @pl.when(c == 0)
        def _():
            edge_loop(src_r, dst_r, h_lo, True)

        @pl.when(c == 1)
        def _():
            edge_loop(src_r, dst_r, h_hi, False)
        plsc.subcore_barrier()

        @pl.when(c == 0)
        def _():
            pltpu.sync_copy(acc.at[pl.ds(row0, ROWS_PER_TILE)],
                            out_l.at[pl.ds(row0, ROWS_PER_TILE)])
            pltpu.sync_copy(cnt_acc.at[pl.ds(row0, ROWS_PER_TILE)],
                            cnt_out.at[pl.ds(row0, ROWS_PER_TILE)])

        @pl.when(c == 1)
        def _():
            pltpu.sync_copy(acc.at[pl.ds(row0, ROWS_PER_TILE)],
                            out_h.at[pl.ds(row0, ROWS_PER_TILE)])
        plsc.subcore_barrier()


@functools.partial(jax.jit, donate_argnums=())
def _sc_sum(h_lo, h_hi, src0, dst0, src1, dst1):
    f32 = jnp.float32
    zrows = jnp.zeros((ROWS_PER_TILE, HH), f32)
    zcnt = jnp.zeros((ROWS_PER_TILE,), f32)
    out_type = [jax.ShapeDtypeStruct((N_PAD, HH), f32) for _ in range(4)] + \
               [jax.ShapeDtypeStruct((N_PAD,), f32) for _ in range(2)]
    scratch = [
        pltpu.VMEM_SHARED((N_PAD, HH), f32),
        pltpu.VMEM_SHARED((N_PAD,), f32),
        pltpu.VMEM((E_PER_TILE,), jnp.int32),
        pltpu.VMEM((E_PER_TILE,), jnp.int32),
        pltpu.VMEM((CH, HH), f32),
        pltpu.VMEM((CH, HH), f32),
        pltpu.VMEM((CH,), f32),
        pltpu.SemaphoreType.DMA,
        pltpu.SemaphoreType.DMA,
    ]
    fn = pl.kernel(
        _sc_sum_body,
        out_type=out_type,
        compiler_params=pltpu.CompilerParams(needs_layout_passes=False),
        mesh=plsc.VectorSubcoreMesh(core_axis_name="c", subcore_axis_name="s"),
        scratch_types=scratch,
    )
    return fn(h_lo, h_hi, src0, dst0, src1, dst1, zrows, zcnt)


def _h_kernel(x_ref, wt_ref, b_ref, h_ref):
    h_ref[...] = jax.nn.relu(
        jnp.dot(x_ref[...], wt_ref[...], preferred_element_type=jnp.float32)
        + b_ref[...]
    )


def _compute_h(x_pad, W_init, b_init):
    wt = W_init.T
    b2 = b_init.reshape(1, H)
    grid = (N_PAD // BLK,)
    return pl.pallas_call(
        _h_kernel,
        grid=grid,
        in_specs=[
            pl.BlockSpec((BLK, H), lambda i: (i, 0)),
            pl.BlockSpec((H, H), lambda i: (0, 0)),
            pl.BlockSpec((1, H), lambda i: (0, 0)),
        ],
        out_specs=pl.BlockSpec((BLK, H), lambda i: (i, 0)),
        out_shape=jax.ShapeDtypeStruct((N_PAD, H), jnp.float32),
    )(x_pad, wt, b2)


def _final_kernel(h_ref, s0l_ref, s0h_ref, m0_ref, s1l_ref, s1h_ref, m1_ref,
                  c0_ref, c1_ref,
                  wlm0_ref, wrm0_ref, wlx0_ref, wrx0_ref,
                  wlm1_ref, wrm1_ref, wlx1_ref, wrx1_ref,
                  wadj_ref, wp0_ref, wp1_ref, wp2_ref, wp3_ref,
                  blm0_ref, blx0_ref, blm1_ref, blx1_ref,
                  badj_ref, bpost_ref, out_ref):
    h = h_ref[...]
    f32 = jnp.float32

    def dot(a, b):
        return jnp.dot(a, b, preferred_element_type=f32)

    inv0 = 1.0 / jnp.maximum(c0_ref[...], 1.0)
    inv1 = 1.0 / jnp.maximum(c1_ref[...], 1.0)

    def mean_dot(sl_ref, sh_ref, inv, w_ref):
        return (dot(sl_ref[...] * inv, w_ref[:HH, :])
                + dot(sh_ref[...] * inv, w_ref[HH:, :]))

    a0 = jax.nn.relu(mean_dot(s0l_ref, s0h_ref, inv0, wlm0_ref)
                     + blm0_ref[...] + dot(h, wrm0_ref[...]))
    b0 = jax.nn.relu(dot(m0_ref[...].astype(f32), wlx0_ref[...])
                     + blx0_ref[...] + dot(h, wrx0_ref[...]))
    a1 = jax.nn.relu(mean_dot(s1l_ref, s1h_ref, inv1, wlm1_ref)
                     + blm1_ref[...] + dot(h, wrm1_ref[...]))
    b1 = jax.nn.relu(dot(m1_ref[...].astype(f32), wlx1_ref[...])
                     + blx1_ref[...] + dot(h, wrx1_ref[...]))

    wadj = wadj_ref[...]
    badj = badj_ref[...]
    ym0 = dot(a0, wadj) + badj
    yx0 = dot(b0, wadj) + badj
    ym1 = dot(a1, wadj) + badj
    yx1 = dot(b1, wadj) + badj

    logits = (dot(ym0, wp0_ref[...]) + dot(yx0, wp1_ref[...])
              + dot(ym1, wp2_ref[...]) + dot(yx1, wp3_ref[...])
              + bpost_ref[...])

    mx = jnp.max(logits, axis=1, keepdims=True)
    z = logits - mx
    lse = jnp.log(jnp.sum(jnp.exp(z), axis=1, keepdims=True))
    out_ref[...] = z - lse


def _final_stage(h_pad, s0l, s0h, m0, s1l, s1h, m1, c0, c1,
                 Wl_mean0, bl_mean0, Wr_mean0, Wl_max0, bl_max0, Wr_max0,
                 Wl_mean1, bl_mean1, Wr_mean1, Wl_max1, bl_max1, Wr_max1,
                 W_adj, b_adj, W_post, b_post):
    grid = (N_PAD // BLK,)
    row_spec = pl.BlockSpec((BLK, H), lambda i: (i, 0))
    half_spec = pl.BlockSpec((BLK, HH), lambda i: (i, 0))
    col_spec = pl.BlockSpec((BLK, 1), lambda i: (i, 0))
    w_spec = pl.BlockSpec((H, H), lambda i: (0, 0))
    b_spec = pl.BlockSpec((1, H), lambda i: (0, 0))
    wp = [W_post[:, k * H:(k + 1) * H].T for k in range(4)]
    return pl.pallas_call(
        _final_kernel,
        grid=grid,
        in_specs=([row_spec, half_spec, half_spec, row_spec,
                   half_spec, half_spec, row_spec]
                  + [col_spec] * 2 + [w_spec] * 13 + [b_spec] * 6),
        out_specs=row_spec,
        out_shape=jax.ShapeDtypeStruct((N_PAD, H), jnp.float32),
    )(h_pad, s0l, s0h, m0, s1l, s1h, m1,
      c0.reshape(N_PAD, 1), c1.reshape(N_PAD, 1),
      Wl_mean0.T, Wr_mean0.T, Wl_max0.T, Wr_max0.T,
      Wl_mean1.T, Wr_mean1.T, Wl_max1.T, Wr_max1.T,
      W_adj.T, wp[0], wp[1], wp[2], wp[3],
      bl_mean0.reshape(1, H), bl_max0.reshape(1, H),
      bl_mean1.reshape(1, H), bl_max1.reshape(1, H),
      b_adj.reshape(1, H), b_post.reshape(1, H))


NW = 32
RNG = N_PAD // NW
TRASH = RNG
HW = 128
SCAN_C = 16000
N_CHUNK = E // SCAN_C
GC = 128
CM_CAP = SCAN_C + GC


def _sc_max_body(hw_hbm, src0, dst0, src1, dst1, zmax,
                 m0, m1,
                 acc, dbuf, sbuf, cm_src, cm_dst, rows0, sg0):
    i32 = jnp.int32
    c = lax.axis_index("c")
    s = lax.axis_index("s")
    w = s * 2 + c
    lo = w * RNG
    iota = lax.iota(i32, 16)
    zeros16 = jnp.zeros((16,), i32)

    def init_cm(i, _):
        cm_src[pl.ds(i * 16, 16)] = zeros16
        return 0
    lax.fori_loop(0, CM_CAP // 16, init_cm, 0)

    def scan_group(g, nm_vec):
        dv = dbuf[pl.ds(g * 16, 16)]
        sv = sbuf[pl.ds(g * 16, 16)]
        dl = dv - lo
        m = (dl >= 0) & (dl < RNG)
        incl = plsc.cumsum(m.astype(i32))
        pos = nm_vec + incl - 1
        plsc.store_scatter(cm_src, [pos], sv, mask=m)
        plsc.store_scatter(cm_dst, [pos], dl, mask=m)
        return nm_vec + plsc.all_reduce_population_count(m)

    def rmw_group(rows_v, off2):
        dlv = cm_dst[pl.ds(off2, 16)]
        for k in range(16):
            dl = dlv[k]
            e = off2 % GC + k
            for f in range(HW // 16):
                a = acc[dl, pl.ds(f * 16, 16)]
                mg = rows_v[e, pl.ds(f * 16, 16)]
                r = jnp.maximum(plsc.bitcast(a, jnp.bfloat16),
                                plsc.bitcast(mg, jnp.bfloat16))
                acc[dl, pl.ds(f * 16, 16)] = plsc.bitcast(r, i32)

    def layer(src_r, dst_r, out_r):
        pltpu.sync_copy(zmax, acc)

        def chunk(j, _):
            off = j * SCAN_C
            pltpu.sync_copy(dst_r.at[pl.ds(off, SCAN_C)], dbuf)
            pltpu.sync_copy(src_r.at[pl.ds(off, SCAN_C)], sbuf)

            def grp4(jj, nm_vec):
                for k in range(4):
                    nm_vec = scan_group(jj * 4 + k, nm_vec)
                return nm_vec
            nm_vec = lax.fori_loop(0, SCAN_C // 64, grp4,
                                   jnp.zeros((16,), i32))
            nm = jnp.max(nm_vec)
            trash_splat = jnp.full((16,), TRASH, i32)
            plsc.store_scatter(cm_dst, [nm + iota], trash_splat)
            nm16 = (nm + 15) // 16 * 16
            n_sub = (nm16 + GC - 1) // GC

            def sub(s2, _):
                pltpu.async_copy(
                    hw_hbm.at[cm_src.at[pl.ds(s2 * GC, GC)]], rows0,
                    sg0).wait()
                ng = jnp.minimum(GC // 16, (nm16 - s2 * GC) // 16)

                def grp(gg, _):
                    rmw_group(rows0, s2 * GC + gg * 16)
                    return 0
                lax.fori_loop(0, ng, grp, 0)
                return 0
            lax.fori_loop(0, n_sub, sub, 0)
            return 0
        lax.fori_loop(0, N_CHUNK, chunk, 0)
        pltpu.sync_copy(acc.at[pl.ds(0, RNG)], out_r.at[pl.ds(lo, RNG)])

    layer(src0, dst0, m0)
    layer(src1, dst1, m1)


def _sc_max(hw, src0, dst0, src1, dst1):
    i32 = jnp.int32
    zmax = jnp.zeros((RNG + 1, HW), i32)
    out_type = [jax.ShapeDtypeStruct((N_PAD, HW), i32) for _ in range(2)]
    scratch = [
        pltpu.VMEM((RNG + 1, HW), i32),
        pltpu.VMEM((SCAN_C,), i32),
        pltpu.VMEM((SCAN_C,), i32),
        pltpu.VMEM((CM_CAP,), i32),
        pltpu.VMEM((CM_CAP,), i32),
        pltpu.VMEM((GC, HW), i32),
        pltpu.SemaphoreType.DMA,
    ]
    fn = pl.kernel(
        _sc_max_body,
        out_type=out_type,
        compiler_params=pltpu.CompilerParams(needs_layout_passes=False),
        mesh=plsc.VectorSubcoreMesh(core_axis_name="c", subcore_axis_name="s"),
        scratch_types=scratch,
    )
    return fn(hw, src0, dst0, src1, dst1, zmax)


def kernel(x, edge_index0, edge_index1, W_init, b_init,
           Wl_mean0, bl_mean0, Wr_mean0, Wl_max0, bl_max0, Wr_max0,
           Wl_mean1, bl_mean1, Wr_mean1, Wl_max1, bl_max1, Wr_max1,
           W_adj, b_adj, W_post, b_post):
    x_pad = jnp.pad(x, ((0, N_PAD - N), (0, 0)))
    h_pad = _compute_h(x_pad, W_init, b_init)
    h_lo = h_pad[:, :HH] + 0.0
    h_hi = h_pad[:, HH:] + 0.0
    hw = jax.lax.bitcast_convert_type(
        h_pad.astype(jnp.bfloat16).reshape(N_PAD, HW, 2), jnp.int32)
    s0l, s0h, s1l, s1h, c0, c1 = _sc_sum(
        h_lo, h_hi, edge_index0[0], edge_index0[1],
        edge_index1[0], edge_index1[1])
    m0w, m1w = _sc_max(hw, edge_index0[0], edge_index0[1],
                       edge_index1[0], edge_index1[1])
    m0 = jax.lax.bitcast_convert_type(m0w, jnp.bfloat16).reshape(N_PAD, H)
    m1 = jax.lax.bitcast_convert_type(m1w, jnp.bfloat16).reshape(N_PAD, H)
    out = _final_stage(h_pad, s0l, s0h, m0, s1l, s1h, m1, c0, c1,
                       Wl_mean0, bl_mean0, Wr_mean0, Wl_max0, bl_max0, Wr_max0,
                       Wl_mean1, bl_mean1, Wr_mean1, Wl_max1, bl_max1, Wr_max1,
                       W_adj, b_adj, W_post, b_post)
    return out[:N]


# speedup vs baseline: 2.1613x; 1.3008x over previous
import functools

import jax
import jax.numpy as jnp
from jax import lax
from jax.experimental import pallas as pl
from jax.experimental.pallas import tpu as pltpu
from jax.experimental.pallas import tpu_sc as plsc

N = 10000
E = 160000
H = 256
BLK = 256
N_PAD = 10240
HH = 128
NT = 16
ROWS_PER_TILE = N_PAD // NT
E_PER_TILE = E // NT
CH = 80
NCH = E_PER_TILE // CH


def _sc_sum_body(h_lo, h_hi, src0, dst0, src1, dst1, zrows, zcnt,
                 s0l, s0h, s1l, s1h, c0, c1,
                 acc, cnt_acc, src_v, dst_v, rows_a, rows_b, ones_v,
                 sga, sgb):
    c = lax.axis_index("c")
    s = lax.axis_index("s")
    for i in range(CH // 16):
        ones_v[pl.ds(i * 16, 16)] = jnp.full((16,), 1.0, jnp.float32)

    def edge_loop(src_r, dst_r, h_tab, count):
        pltpu.sync_copy(src_r.at[pl.ds(s * E_PER_TILE, E_PER_TILE)], src_v)
        pltpu.sync_copy(dst_r.at[pl.ds(s * E_PER_TILE, E_PER_TILE)], dst_v)
        rows = (rows_a, rows_b)
        sems = (sga, sgb)
        pltpu.async_copy(h_tab.at[src_v.at[pl.ds(0, CH)]], rows_a, sga)

        def do(j, b):
            @pl.when(j + 1 < NCH)
            def _():
                pltpu.async_copy(
                    h_tab.at[src_v.at[pl.ds((j + 1) * CH, CH)]],
                    rows[1 - b], sems[1 - b])
            pltpu.make_async_copy(
                h_tab.at[src_v.at[pl.ds(j * CH, CH)]], rows[b],
                sems[b]).wait()
            pltpu.sync_copy(rows[b], acc.at[dst_v.at[pl.ds(j * CH, CH)]],
                            add=True)
            if count:
                pltpu.sync_copy(ones_v,
                                cnt_acc.at[dst_v.at[pl.ds(j * CH, CH)]],
                                add=True)

        def pair(jj, carry):
            do(jj * 2, 0)
            do(jj * 2 + 1, 1)
            return carry
        lax.fori_loop(0, NCH // 2, pair, 0)
        do(NCH - 1, 0)

    row0 = s * ROWS_PER_TILE
    for (src_r, dst_r, out_l, out_h, cnt_out) in (
            (src0, dst0, s0l, s0h, c0), (src1, dst1, s1l, s1h, c1)):
        pltpu.sync_copy(zrows, acc.at[pl.ds(row0, ROWS_PER_TILE)])

        @pl.when(c == 0)
        def _():
            pltpu.sync_copy(zcnt, cnt_acc.at[pl.ds(row0, ROWS_PER_TILE)])
        plsc.subcore_barrier()

        @pl.when(c == 0)
        def _():
            edge_loop(src_r, dst_r, h_lo, True)

        @pl.when(c == 1)
        def _():
            edge_loop(src_r, dst_r, h_hi, False)
        plsc.subcore_barrier()

        @pl.when(c == 0)
        def _():
            pltpu.sync_copy(acc.at[pl.ds(row0, ROWS_PER_TILE)],
                            out_l.at[pl.ds(row0, ROWS_PER_TILE)])
            pltpu.sync_copy(cnt_acc.at[pl.ds(row0, ROWS_PER_TILE)],
                            cnt_out.at[pl.ds(row0, ROWS_PER_TILE)])

        @pl.when(c == 1)
        def _():
            pltpu.sync_copy(acc.at[pl.ds(row0, ROWS_PER_TILE)],
                            out_h.at[pl.ds(row0, ROWS_PER_TILE)])
        plsc.subcore_barrier()


@functools.partial(jax.jit, donate_argnums=())
def _sc_sum(h_lo, h_hi, src0, dst0, src1, dst1):
    f32 = jnp.float32
    zrows = jnp.zeros((ROWS_PER_TILE, HH), f32)
    zcnt = jnp.zeros((ROWS_PER_TILE,), f32)
    out_type = [jax.ShapeDtypeStruct((N_PAD, HH), f32) for _ in range(4)] + \
               [jax.ShapeDtypeStruct((N_PAD,), f32) for _ in range(2)]
    scratch = [
        pltpu.VMEM_SHARED((N_PAD, HH), f32),
        pltpu.VMEM_SHARED((N_PAD,), f32),
        pltpu.VMEM((E_PER_TILE,), jnp.int32),
        pltpu.VMEM((E_PER_TILE,), jnp.int32),
        pltpu.VMEM((CH, HH), f32),
        pltpu.VMEM((CH, HH), f32),
        pltpu.VMEM((CH,), f32),
        pltpu.SemaphoreType.DMA,
        pltpu.SemaphoreType.DMA,
    ]
    fn = pl.kernel(
        _sc_sum_body,
        out_type=out_type,
        compiler_params=pltpu.CompilerParams(needs_layout_passes=False),
        mesh=plsc.VectorSubcoreMesh(core_axis_name="c", subcore_axis_name="s"),
        scratch_types=scratch,
    )
    return fn(h_lo, h_hi, src0, dst0, src1, dst1, zrows, zcnt)


def _h_kernel(x_ref, wt_ref, b_ref, h_ref):
    h_ref[...] = jax.nn.relu(
        jnp.dot(x_ref[...], wt_ref[...], preferred_element_type=jnp.float32)
        + b_ref[...]
    )


def _compute_h(x_pad, W_init, b_init):
    wt = W_init.T
    b2 = b_init.reshape(1, H)
    grid = (N_PAD // BLK,)
    return pl.pallas_call(
        _h_kernel,
        grid=grid,
        in_specs=[
            pl.BlockSpec((BLK, H), lambda i: (i, 0)),
            pl.BlockSpec((H, H), lambda i: (0, 0)),
            pl.BlockSpec((1, H), lambda i: (0, 0)),
        ],
        out_specs=pl.BlockSpec((BLK, H), lambda i: (i, 0)),
        out_shape=jax.ShapeDtypeStruct((N_PAD, H), jnp.float32),
    )(x_pad, wt, b2)


def _final_kernel(h_ref, s0l_ref, s0h_ref, m0_ref, s1l_ref, s1h_ref, m1_ref,
                  c0_ref, c1_ref,
                  wlm0_ref, wrm0_ref, wlx0_ref, wrx0_ref,
                  wlm1_ref, wrm1_ref, wlx1_ref, wrx1_ref,
                  wadj_ref, wp0_ref, wp1_ref, wp2_ref, wp3_ref,
                  blm0_ref, blx0_ref, blm1_ref, blx1_ref,
                  badj_ref, bpost_ref, out_ref):
    h = h_ref[...]
    f32 = jnp.float32

    def dot(a, b):
        return jnp.dot(a, b, preferred_element_type=f32)

    inv0 = 1.0 / jnp.maximum(c0_ref[...], 1.0)
    inv1 = 1.0 / jnp.maximum(c1_ref[...], 1.0)

    def mean_dot(sl_ref, sh_ref, inv, w_ref):
        return (dot(sl_ref[...] * inv, w_ref[:HH, :])
                + dot(sh_ref[...] * inv, w_ref[HH:, :]))

    a0 = jax.nn.relu(mean_dot(s0l_ref, s0h_ref, inv0, wlm0_ref)
                     + blm0_ref[...] + dot(h, wrm0_ref[...]))
    b0 = jax.nn.relu(dot(m0_ref[...].astype(f32), wlx0_ref[...])
                     + blx0_ref[...] + dot(h, wrx0_ref[...]))
    a1 = jax.nn.relu(mean_dot(s1l_ref, s1h_ref, inv1, wlm1_ref)
                     + blm1_ref[...] + dot(h, wrm1_ref[...]))
    b1 = jax.nn.relu(dot(m1_ref[...].astype(f32), wlx1_ref[...])
                     + blx1_ref[...] + dot(h, wrx1_ref[...]))

    wadj = wadj_ref[...]
    badj = badj_ref[...]
    ym0 = dot(a0, wadj) + badj
    yx0 = dot(b0, wadj) + badj
    ym1 = dot(a1, wadj) + badj
    yx1 = dot(b1, wadj) + badj

    logits = (dot(ym0, wp0_ref[...]) + dot(yx0, wp1_ref[...])
              + dot(ym1, wp2_ref[...]) + dot(yx1, wp3_ref[...])
              + bpost_ref[...])

    mx = jnp.max(logits, axis=1, keepdims=True)
    z = logits - mx
    lse = jnp.log(jnp.sum(jnp.exp(z), axis=1, keepdims=True))
    out_ref[...] = z - lse


def _final_stage(h_pad, s0l, s0h, m0, s1l, s1h, m1, c0, c1,
                 Wl_mean0, bl_mean0, Wr_mean0, Wl_max0, bl_max0, Wr_max0,
                 Wl_mean1, bl_mean1, Wr_mean1, Wl_max1, bl_max1, Wr_max1,
                 W_adj, b_adj, W_post, b_post):
    grid = (N_PAD // BLK,)
    row_spec = pl.BlockSpec((BLK, H), lambda i: (i, 0))
    half_spec = pl.BlockSpec((BLK, HH), lambda i: (i, 0))
    col_spec = pl.BlockSpec((BLK, 1), lambda i: (i, 0))
    w_spec = pl.BlockSpec((H, H), lambda i: (0, 0))
    b_spec = pl.BlockSpec((1, H), lambda i: (0, 0))
    wp = [W_post[:, k * H:(k + 1) * H].T for k in range(4)]
    return pl.pallas_call(
        _final_kernel,
        grid=grid,
        in_specs=([row_spec, half_spec, half_spec, row_spec,
                   half_spec, half_spec, row_spec]
                  + [col_spec] * 2 + [w_spec] * 13 + [b_spec] * 6),
        out_specs=row_spec,
        out_shape=jax.ShapeDtypeStruct((N_PAD, H), jnp.float32),
    )(h_pad, s0l, s0h, m0, s1l, s1h, m1,
      c0.reshape(N_PAD, 1), c1.reshape(N_PAD, 1),
      Wl_mean0.T, Wr_mean0.T, Wl_max0.T, Wr_max0.T,
      Wl_mean1.T, Wr_mean1.T, Wl_max1.T, Wr_max1.T,
      W_adj.T, wp[0], wp[1], wp[2], wp[3],
      bl_mean0.reshape(1, H), bl_max0.reshape(1, H),
      bl_mean1.reshape(1, H), bl_max1.reshape(1, H),
      b_adj.reshape(1, H), b_post.reshape(1, H))


NW = 32
RNG = N_PAD // NW
TRASH = RNG
HW = 128
SCAN_C = 16000
N_CHUNK = E // SCAN_C
GC = 64
CM_CAP = SCAN_C + GC


def _sc_max_body(hw_hbm, src0, dst0, src1, dst1, zmax,
                 m0, m1,
                 acc, dbuf, sbuf, cm_src, cm_dst, rows0, sg0):
    i32 = jnp.int32
    c = lax.axis_index("c")
    s = lax.axis_index("s")
    w = s * 2 + c
    lo = w * RNG
    iota = lax.iota(i32, 16)
    zeros16 = jnp.zeros((16,), i32)

    def init_cm(i, _):
        cm_src[pl.ds(i * 16, 16)] = zeros16
        return 0
    lax.fori_loop(0, CM_CAP // 16, init_cm, 0)

    def scan_group(g, nm_vec):
        dv = dbuf[pl.ds(g * 16, 16)]
        sv = sbuf[pl.ds(g * 16, 16)]
        dl = dv - lo
        m = (dl >= 0) & (dl < RNG)
        incl = plsc.cumsum(m.astype(i32))
        pos = nm_vec + incl - 1
        plsc.store_scatter(cm_src, [pos], sv, mask=m)
        plsc.store_scatter(cm_dst, [pos], dl, mask=m)
        return nm_vec + plsc.all_reduce_population_count(m)

    def rmw_group(rows_v, off2):
        dlv = cm_dst[pl.ds(off2, 16)]
        for k in range(16):
            dl = dlv[k]
            e = off2 % GC + k
            for f in range(HW // 16):
                a = acc[dl, pl.ds(f * 16, 16)]
                mg = rows_v[e, pl.ds(f * 16, 16)]
                r = jnp.maximum(plsc.bitcast(a, jnp.bfloat16),
                                plsc.bitcast(mg, jnp.bfloat16))
                acc[dl, pl.ds(f * 16, 16)] = plsc.bitcast(r, i32)

    def layer(src_r, dst_r, out_r):
        pltpu.sync_copy(zmax, acc)

        def chunk(j, _):
            off = j * SCAN_C
            pltpu.sync_copy(dst_r.at[pl.ds(off, SCAN_C)], dbuf)
            pltpu.sync_copy(src_r.at[pl.ds(off, SCAN_C)], sbuf)

            def grp4(jj, nm_vec):
                for k in range(4):
                    nm_vec = scan_group(jj * 4 + k, nm_vec)
                return nm_vec
            nm_vec = lax.fori_loop(0, SCAN_C // 64, grp4,
                                   jnp.zeros((16,), i32))
            nm = jnp.max(nm_vec)
            trash_splat = jnp.full((16,), TRASH, i32)
            plsc.store_scatter(cm_dst, [nm + iota], trash_splat)
            nm16 = (nm + 15) // 16 * 16
            n_sub = (nm16 + GC - 1) // GC

            def sub(s2, _):
                pltpu.async_copy(
                    hw_hbm.at[cm_src.at[pl.ds(s2 * GC, GC)]], rows0,
                    sg0).wait()
                ng = jnp.minimum(GC // 16, (nm16 - s2 * GC) // 16)

                def grp(gg, _):
                    rmw_group(rows0, s2 * GC + gg * 16)
                    return 0
                lax.fori_loop(0, ng, grp, 0)
                return 0
            lax.fori_loop(0, n_sub, sub, 0)
            return 0
        lax.fori_loop(0, N_CHUNK, chunk, 0)
        pltpu.sync_copy(acc.at[pl.ds(0, RNG)], out_r.at[pl.ds(lo, RNG)])

    layer(src0, dst0, m0)
    layer(src1, dst1, m1)


def _sc_max(hw, src0, dst0, src1, dst1):
    i32 = jnp.int32
    zmax = jnp.zeros((RNG + 1, HW), i32)
    out_type = [jax.ShapeDtypeStruct((N_PAD, HW), i32) for _ in range(2)]
    scratch = [
        pltpu.VMEM((RNG + 1, HW), i32),
        pltpu.VMEM((SCAN_C,), i32),
        pltpu.VMEM((SCAN_C,), i32),
        pltpu.VMEM((CM_CAP,), i32),
        pltpu.VMEM((CM_CAP,), i32),
        pltpu.VMEM((GC, HW), i32),
        pltpu.SemaphoreType.DMA,
    ]
    fn = pl.kernel(
        _sc_max_body,
        out_type=out_type,
        compiler_params=pltpu.CompilerParams(needs_layout_passes=False),
        mesh=plsc.VectorSubcoreMesh(core_axis_name="c", subcore_axis_name="s"),
        scratch_types=scratch,
    )
    return fn(hw, src0, dst0, src1, dst1, zmax)


def kernel(x, edge_index0, edge_index1, W_init, b_init,
           Wl_mean0, bl_mean0, Wr_mean0, Wl_max0, bl_max0, Wr_max0,
           Wl_mean1, bl_mean1, Wr_mean1, Wl_max1, bl_max1, Wr_max1,
           W_adj, b_adj, W_post, b_post):
    x_pad = jnp.pad(x, ((0, N_PAD - N), (0, 0)))
    h_pad = _compute_h(x_pad, W_init, b_init)
    h_lo = h_pad[:, :HH] + 0.0
    h_hi = h_pad[:, HH:] + 0.0
    hw = jax.lax.bitcast_convert_type(
        h_pad.astype(jnp.bfloat16).reshape(N_PAD, HW, 2), jnp.int32)
    s0l, s0h, s1l, s1h, c0, c1 = _sc_sum(
        h_lo, h_hi, edge_index0[0], edge_index0[1],
        edge_index1[0], edge_index1[1])
    m0w, m1w = _sc_max(hw, edge_index0[0], edge_index0[1],
                       edge_index1[0], edge_index1[1])
    m0 = jax.lax.bitcast_convert_type(m0w, jnp.bfloat16).reshape(N_PAD, H)
    m1 = jax.lax.bitcast_convert_type(m1w, jnp.bfloat16).reshape(N_PAD, H)
    out = _final_stage(h_pad, s0l, s0h, m0, s1l, s1h, m1, c0, c1,
                       Wl_mean0, bl_mean0, Wr_mean0, Wl_max0, bl_max0, Wr_max0,
                       Wl_mean1, bl_mean1, Wr_mean1, Wl_max1, bl_max1, Wr_max1,
                       W_adj, b_adj, W_post, b_post)
    return out[:N]
